# baseline (speedup 1.0000x reference)
import functools

import jax
import jax.numpy as jnp
from jax import lax
from jax.experimental import pallas as pl
from jax.experimental.pallas import tpu as pltpu
from jax.experimental.pallas import tpu_sc as plsc

BATCH = 4096
SEQ = 200
EMBED_DIM = 64

NUM_CORES = 2
NUM_SUBCORES = 16
NW = NUM_CORES * NUM_SUBCORES
B_PER_W = BATCH // NW
CHUNK = 40
CHUNKS_PER_SAMPLE = SEQ // CHUNK
CHUNKS_PER_W = B_PER_W * CHUNKS_PER_SAMPLE
NREG = EMBED_DIM // 16


def _pool_body(x_hbm, emb_hbm, out_hbm, idx_v, rows_v, stage_v, sem):
    wid = lax.axis_index("s") * NUM_CORES + lax.axis_index("c")
    pltpu.sync_copy(x_hbm.at[pl.ds(wid * CHUNKS_PER_W, CHUNKS_PER_W), :], idx_v)

    def sample_body(s, carry):
        copies = []
        for j in range(CHUNKS_PER_SAMPLE):
            copies.append(
                pltpu.async_copy(
                    emb_hbm.at[idx_v.at[s * CHUNKS_PER_SAMPLE + j]],
                    rows_v.at[pl.ds(j * CHUNK, CHUNK)],
                    sem,
                )
            )
        for c in copies:
            c.wait()

        def acc_body(r, acc):
            return tuple(
                acc[c] + rows_v[r, pl.ds(c * 16, 16)] for c in range(NREG)
            )

        zeros = tuple(jnp.zeros((16,), jnp.float32) for _ in range(NREG))
        acc = lax.fori_loop(0, SEQ, acc_body, zeros)
        for c in range(NREG):
            stage_v[s, pl.ds(c * 16, 16)] = acc[c]
        return carry

    lax.fori_loop(0, B_PER_W, sample_body, 0)
    pltpu.sync_copy(stage_v, out_hbm.at[pl.ds(wid * B_PER_W, B_PER_W), :])


@jax.jit
def _pool(x_r, emb):
    mesh = plsc.VectorSubcoreMesh(
        core_axis_name="c",
        subcore_axis_name="s",
        num_cores=NUM_CORES,
        num_subcores=NUM_SUBCORES,
    )
    return pl.kernel(
        _pool_body,
        out_type=jax.ShapeDtypeStruct((BATCH, EMBED_DIM), jnp.float32),
        mesh=mesh,
        scratch_types=[
            pltpu.VMEM((CHUNKS_PER_W, CHUNK), jnp.int32),
            pltpu.VMEM((SEQ, EMBED_DIM), jnp.float32),
            pltpu.VMEM((B_PER_W, EMBED_DIM), jnp.float32),
            pltpu.SemaphoreType.DMA,
        ],
        compiler_params=pltpu.CompilerParams(use_tc_tiling_on_sc=False),
    )(x_r, emb)


def _mlp_body(h_ref, w1_ref, b1_ref, w2_ref, b2_ref, o_ref):
    h = h_ref[...] * (1.0 / SEQ)
    z = jnp.dot(h, w1_ref[...], preferred_element_type=jnp.float32) + b1_ref[...]
    z = jnp.maximum(z, 0.0)
    o_ref[...] = jnp.dot(z, w2_ref[...], preferred_element_type=jnp.float32) + b2_ref[...]


@jax.jit
def _mlp(pooled, W1, b1, W2, b2):
    return pl.pallas_call(
        _mlp_body,
        out_shape=jax.ShapeDtypeStruct((BATCH, 1), jnp.float32),
    )(pooled, W1, b1.reshape(1, 32), W2, b2.reshape(1, 1))


def kernel(x, emb, W1, b1, W2, b2):
    x_r = x.reshape(BATCH * CHUNKS_PER_SAMPLE, CHUNK).astype(jnp.int32)
    pooled = _pool(x_r, emb)
    return _mlp(pooled, W1, b1, W2, b2)


# speedup vs baseline: 1.1674x; 1.1674x over previous
import functools

import jax
import jax.numpy as jnp
from jax import lax
from jax.experimental import pallas as pl
from jax.experimental.pallas import tpu as pltpu
from jax.experimental.pallas import tpu_sc as plsc

BATCH = 4096
SEQ = 200
EMBED_DIM = 64

NUM_CORES = 2
NUM_SUBCORES = 16
NW = NUM_CORES * NUM_SUBCORES
B_PER_W = BATCH // NW
CHUNK = 40
CHUNKS_PER_SAMPLE = SEQ // CHUNK
CHUNKS_PER_W = B_PER_W * CHUNKS_PER_SAMPLE
NREG = EMBED_DIM // 16


UNROLL = 8


def _pool_body(x_hbm, emb_hbm, out_hbm, idx_v, rows0_v, rows1_v, stage_v, sem0, sem1):
    wid = lax.axis_index("s") * NUM_CORES + lax.axis_index("c")
    pltpu.sync_copy(x_hbm.at[pl.ds(wid * CHUNKS_PER_W, CHUNKS_PER_W), :], idx_v)

    bufs = ((rows0_v, sem0), (rows1_v, sem1))

    def issue(s, rv, sem):
        for j in range(CHUNKS_PER_SAMPLE):
            pltpu.async_copy(
                emb_hbm.at[idx_v.at[s * CHUNKS_PER_SAMPLE + j]],
                rv.at[pl.ds(j * CHUNK, CHUNK)],
                sem,
            )

    def wait(rv, sem):
        pltpu.make_async_copy(emb_hbm.at[pl.ds(0, SEQ), :], rv, sem).wait()

    def accumulate(s, rv):
        def acc_body(i, acc):
            acc = list(acc)
            for u in range(UNROLL):
                r = i * UNROLL + u
                for c in range(NREG):
                    acc[c] = acc[c] + rv[r, pl.ds(c * 16, 16)]
            return tuple(acc)

        zeros = tuple(jnp.zeros((16,), jnp.float32) for _ in range(NREG))
        acc = lax.fori_loop(0, SEQ // UNROLL, acc_body, zeros)
        for c in range(NREG):
            stage_v[s, pl.ds(c * 16, 16)] = acc[c]

    issue(0, rows0_v, sem0)
    issue(1, rows1_v, sem1)

    def body(t, carry):
        for b, (rv, sem) in enumerate(bufs):
            s = 2 * t + b
            wait(rv, sem)
            accumulate(s, rv)
            issue(s + 2, rv, sem)
        return carry

    lax.fori_loop(0, B_PER_W // 2 - 1, body, 0)
    for b, (rv, sem) in enumerate(bufs):
        s = B_PER_W - 2 + b
        wait(rv, sem)
        accumulate(s, rv)

    pltpu.sync_copy(stage_v, out_hbm.at[pl.ds(wid * B_PER_W, B_PER_W), :])


@jax.jit
def _pool(x_r, emb):
    mesh = plsc.VectorSubcoreMesh(
        core_axis_name="c",
        subcore_axis_name="s",
        num_cores=NUM_CORES,
        num_subcores=NUM_SUBCORES,
    )
    return pl.kernel(
        _pool_body,
        out_type=jax.ShapeDtypeStruct((BATCH, EMBED_DIM), jnp.float32),
        mesh=mesh,
        scratch_types=[
            pltpu.VMEM((CHUNKS_PER_W, CHUNK), jnp.int32),
            pltpu.VMEM((SEQ, EMBED_DIM), jnp.float32),
            pltpu.VMEM((SEQ, EMBED_DIM), jnp.float32),
            pltpu.VMEM((B_PER_W, EMBED_DIM), jnp.float32),
            pltpu.SemaphoreType.DMA,
            pltpu.SemaphoreType.DMA,
        ],
        compiler_params=pltpu.CompilerParams(use_tc_tiling_on_sc=False),
    )(x_r, emb)


def _mlp_body(h_ref, w1_ref, b1_ref, w2_ref, b2_ref, o_ref):
    h = h_ref[...] * (1.0 / SEQ)
    z = jnp.dot(h, w1_ref[...], preferred_element_type=jnp.float32) + b1_ref[...]
    z = jnp.maximum(z, 0.0)
    o_ref[...] = jnp.dot(z, w2_ref[...], preferred_element_type=jnp.float32) + b2_ref[...]


@jax.jit
def _mlp(pooled, W1, b1, W2, b2):
    return pl.pallas_call(
        _mlp_body,
        out_shape=jax.ShapeDtypeStruct((BATCH, 1), jnp.float32),
    )(pooled, W1, b1.reshape(1, 32), W2, b2.reshape(1, 1))


def kernel(x, emb, W1, b1, W2, b2):
    x_r = x.reshape(BATCH * CHUNKS_PER_SAMPLE, CHUNK).astype(jnp.int32)
    pooled = _pool(x_r, emb)
    return _mlp(pooled, W1, b1, W2, b2)


# speedup vs baseline: 1.2007x; 1.0286x over previous
import jax
import jax.numpy as jnp
from jax import lax
from jax.experimental import pallas as pl
from jax.experimental.pallas import tpu as pltpu
from jax.experimental.pallas import tpu_sc as plsc

BATCH = 4096
SEQ = 200
EMBED_DIM = 64

NUM_CORES = 2
NUM_SUBCORES = 16
NW = NUM_CORES * NUM_SUBCORES
B_PER_W = BATCH // NW
IDX_PER_W = B_PER_W * SEQ
GROUP = 2
GROUP_ROWS = GROUP * SEQ
NREG = EMBED_DIM // 16
UNROLL = 8


def _pool_body(x_hbm, emb_hbm, out_hbm, idx_v, rows0_v, rows1_v, stage_v, sem0, sem1):
    wid = lax.axis_index("s") * NUM_CORES + lax.axis_index("c")
    pltpu.sync_copy(x_hbm.at[pl.ds(wid * IDX_PER_W, IDX_PER_W)], idx_v)

    bufs = ((rows0_v, sem0), (rows1_v, sem1))

    def issue(g, rv, sem):
        pltpu.async_copy(
            emb_hbm.at[idx_v.at[pl.ds(g * GROUP_ROWS, GROUP_ROWS)]],
            rv,
            sem,
        )

    def wait(rv, sem):
        pltpu.make_async_copy(emb_hbm.at[pl.ds(0, GROUP_ROWS), :], rv, sem).wait()

    def accumulate(g, rv):
        def acc_body(i, acc):
            acc = list(acc)
            for u in range(UNROLL):
                r = i * UNROLL + u
                for k in range(GROUP):
                    for c in range(NREG):
                        j = k * NREG + c
                        acc[j] = acc[j] + rv[k * SEQ + r, pl.ds(c * 16, 16)]
            return tuple(acc)

        zeros = tuple(jnp.zeros((16,), jnp.float32) for _ in range(GROUP * NREG))
        acc = lax.fori_loop(0, SEQ // UNROLL, acc_body, zeros)
        for k in range(GROUP):
            for c in range(NREG):
                stage_v[g * GROUP + k, pl.ds(c * 16, 16)] = acc[k * NREG + c]

    n_groups = B_PER_W // GROUP
    issue(0, rows0_v, sem0)
    issue(1, rows1_v, sem1)

    def body(t, carry):
        for b, (rv, sem) in enumerate(bufs):
            g = 2 * t + b
            wait(rv, sem)
            accumulate(g, rv)
            issue(g + 2, rv, sem)
        return carry

    lax.fori_loop(0, n_groups // 2 - 1, body, 0)
    for b, (rv, sem) in enumerate(bufs):
        g = n_groups - 2 + b
        wait(rv, sem)
        accumulate(g, rv)

    pltpu.sync_copy(stage_v, out_hbm.at[pl.ds(wid * B_PER_W, B_PER_W), :])


@jax.jit
def _pool(x_flat, emb):
    mesh = plsc.VectorSubcoreMesh(
        core_axis_name="c",
        subcore_axis_name="s",
        num_cores=NUM_CORES,
        num_subcores=NUM_SUBCORES,
    )
    return pl.kernel(
        _pool_body,
        out_type=jax.ShapeDtypeStruct((BATCH, EMBED_DIM), jnp.float32),
        mesh=mesh,
        scratch_types=[
            pltpu.VMEM((IDX_PER_W,), jnp.int32),
            pltpu.VMEM((GROUP_ROWS, EMBED_DIM), jnp.float32),
            pltpu.VMEM((GROUP_ROWS, EMBED_DIM), jnp.float32),
            pltpu.VMEM((B_PER_W, EMBED_DIM), jnp.float32),
            pltpu.SemaphoreType.DMA,
            pltpu.SemaphoreType.DMA,
        ],
        compiler_params=pltpu.CompilerParams(use_tc_tiling_on_sc=False),
    )(x_flat, emb)


def _mlp_body(h_ref, w1_ref, b1_ref, w2_ref, b2_ref, o_ref):
    h = h_ref[...] * (1.0 / SEQ)
    z = jnp.dot(h, w1_ref[...], preferred_element_type=jnp.float32) + b1_ref[...]
    z = jnp.maximum(z, 0.0)
    o_ref[...] = jnp.dot(z, w2_ref[...], preferred_element_type=jnp.float32) + b2_ref[...]


@jax.jit
def _mlp(pooled, W1, b1, W2, b2):
    return pl.pallas_call(
        _mlp_body,
        out_shape=jax.ShapeDtypeStruct((BATCH, 1), jnp.float32),
    )(pooled, W1, b1.reshape(1, 32), W2, b2.reshape(1, 1))


def kernel(x, emb, W1, b1, W2, b2):
    x_flat = x.reshape(BATCH * SEQ).astype(jnp.int32)
    pooled = _pool(x_flat, emb)
    return _mlp(pooled, W1, b1, W2, b2)


# speedup vs baseline: 1.3229x; 1.1018x over previous
import jax
import jax.numpy as jnp
from jax import lax
from jax.experimental import pallas as pl
from jax.experimental.pallas import tpu as pltpu
from jax.experimental.pallas import tpu_sc as plsc

BATCH = 4096
SEQ = 200
EMBED_DIM = 64

NUM_CORES = 2
NUM_SUBCORES = 16
NW = NUM_CORES * NUM_SUBCORES
B_PER_W = BATCH // NW
IDX_PER_W = B_PER_W * SEQ
GROUP = 2
GROUP_ROWS = GROUP * SEQ
NREG = EMBED_DIM // 16
UNROLL = 8


def _pool_body(x_hbm, emb_hbm, out_hbm, idx_v, rows0_v, rows1_v, stage_v, sem0, sem1):
    wid = lax.axis_index("s") * NUM_CORES + lax.axis_index("c")
    pltpu.sync_copy(x_hbm.at[pl.ds(wid * IDX_PER_W, IDX_PER_W)], idx_v)

    bufs = ((rows0_v, sem0), (rows1_v, sem1))

    def issue(g, rv, sem):
        pltpu.async_copy(
            emb_hbm.at[idx_v.at[pl.ds(g * GROUP_ROWS, GROUP_ROWS)]],
            rv,
            sem,
        )

    def wait(rv, sem):
        pltpu.make_async_copy(emb_hbm.at[pl.ds(0, GROUP_ROWS), :], rv, sem).wait()

    def accumulate(g, rv):
        def acc_body(i, acc):
            acc = list(acc)
            for u in range(UNROLL):
                r = i * UNROLL + u
                for k in range(GROUP):
                    for c in range(NREG):
                        j = k * NREG + c
                        acc[j] = acc[j] + rv[k * SEQ + r, pl.ds(c * 16, 16)]
            return tuple(acc)

        zeros = tuple(jnp.zeros((16,), jnp.float32) for _ in range(GROUP * NREG))
        acc = lax.fori_loop(0, SEQ // UNROLL, acc_body, zeros)
        for k in range(GROUP):
            for c in range(NREG):
                stage_v[g * GROUP + k, pl.ds(c * 16, 16)] = acc[k * NREG + c]

    n_groups = B_PER_W // GROUP
    issue(0, rows0_v, sem0)
    issue(1, rows1_v, sem1)

    def body(t, carry):
        for b, (rv, sem) in enumerate(bufs):
            g = 2 * t + b
            wait(rv, sem)
            accumulate(g, rv)
            issue(g + 2, rv, sem)
        return carry

    lax.fori_loop(0, n_groups // 2 - 1, body, 0)
    for b, (rv, sem) in enumerate(bufs):
        g = n_groups - 2 + b
        wait(rv, sem)
        accumulate(g, rv)

    pltpu.sync_copy(stage_v, out_hbm.at[pl.ds(wid * B_PER_W, B_PER_W), :])


@jax.jit
def _pool(x_flat, emb):
    mesh = plsc.VectorSubcoreMesh(
        core_axis_name="c",
        subcore_axis_name="s",
        num_cores=NUM_CORES,
        num_subcores=NUM_SUBCORES,
    )
    return pl.kernel(
        _pool_body,
        out_type=jax.ShapeDtypeStruct((BATCH, EMBED_DIM), jnp.float32),
        mesh=mesh,
        scratch_types=[
            pltpu.VMEM((IDX_PER_W,), jnp.int32),
            pltpu.VMEM((GROUP_ROWS, EMBED_DIM), jnp.float32),
            pltpu.VMEM((GROUP_ROWS, EMBED_DIM), jnp.float32),
            pltpu.VMEM((B_PER_W, EMBED_DIM), jnp.float32),
            pltpu.SemaphoreType.DMA,
            pltpu.SemaphoreType.DMA,
        ],
        compiler_params=pltpu.CompilerParams(use_tc_tiling_on_sc=False),
    )(x_flat, emb)


def _mlp_body(h_ref, w1_ref, b1_ref, w2_ref, b2_ref, o_ref):
    h = h_ref[...] * (1.0 / SEQ)
    z = jnp.dot(h, w1_ref[...], preferred_element_type=jnp.float32) + b1_ref[...]
    z = jnp.maximum(z, 0.0)
    o_ref[...] = jnp.dot(z, w2_ref[...], preferred_element_type=jnp.float32) + b2_ref[...]


@jax.jit
def _mlp(pooled, W1, b1, W2, b2):
    return pl.pallas_call(
        _mlp_body,
        out_shape=jax.ShapeDtypeStruct((BATCH, 1), jnp.float32),
    )(pooled, W1, b1.reshape(1, 32), W2, b2.reshape(1, 1))


def kernel(x, emb, W1, b1, W2, b2):
    x_flat = (x.reshape(BATCH * SEQ) * 2).astype(jnp.int32)
    vocab = emb.shape[0]
    emb_pad = jnp.concatenate(
        [emb, jnp.zeros((vocab, EMBED_DIM), jnp.float32)], axis=1
    )
    emb2 = emb_pad.reshape(2 * vocab, EMBED_DIM)
    pooled = _pool(x_flat, emb2)
    return _mlp(pooled, W1, b1, W2, b2)
